# baseline (speedup 1.0000x reference)
import jax
import jax.numpy as jnp
from jax import lax
from jax.experimental import pallas as pl
from jax.experimental.pallas import tpu as pltpu

_EPSILON = 1e-06
_SOFT_MARGIN = 0.5
_HARD_MARGIN = 0.1
_NUM_NEG = 8
_B = 16
_D = 128
_T = 4096
_NSTREAM = 4
_CH = _T // _NSTREAM


def _chunk_loss(ctx, prin, act, rid):
    logits = lax.dot_general(ctx, act,
                             dimension_numbers=(((1,), (1,)), ((), ())),
                             preferred_element_type=jnp.float32)
    logits = logits * (1.0 / (_D ** 0.5))
    sc = jax.nn.sigmoid(logits)
    s = -jnp.log(_EPSILON + 1.0 - sc)

    row = lax.broadcasted_iota(jnp.int32, (_B, _CH), 0)
    off = (row - rid) & (_B - 1)

    pos = off == 0
    neg = (off >= 1) & (off <= _NUM_NEG)

    s_pos = jnp.sum(jnp.where(pos, s, 0.0), axis=0, keepdims=True)
    prin_q = jnp.sum(jnp.where(pos, prin, 0), axis=0, keepdims=True)
    w = jnp.where(neg & (prin != prin_q), 1.0, 0.0)

    x = s - s_pos + _HARD_MARGIN
    quad = x * x / (2.0 * _SOFT_MARGIN)
    lin = x - _SOFT_MARGIN / 2.0
    h = jnp.where(x <= 0.0, 0.0, jnp.where(x < _SOFT_MARGIN, quad, lin))

    num = jnp.sum(h * w, axis=0, keepdims=True)
    den = jnp.sum(w, axis=0, keepdims=True) + _EPSILON
    per_query = num / den
    return jnp.sum(per_query, axis=1, keepdims=True)


def _loss_kernel(ctx_ref, prin_ref, act_hbm, rowid_ref, out_ref,
                 act_vmem, *sems):
    copies = []
    for c in range(_NSTREAM):
        cp = pltpu.make_async_copy(
            act_hbm.at[pl.ds(c * _CH, _CH), :],
            act_vmem.at[pl.ds(c * _CH, _CH), :],
            sems[c])
        cp.start()
        copies.append(cp)

    ctx = ctx_ref[...]
    prin = prin_ref[...]
    acc = jnp.zeros((1, 1), jnp.float32)
    for c in range(_NSTREAM):
        copies[c].wait()
        act = act_vmem[pl.ds(c * _CH, _CH), :]
        rid = rowid_ref[:, pl.ds(c * _CH, _CH)]
        acc = acc + _chunk_loss(ctx, prin, act, rid)
    out_ref[...] = acc * (1.0 / (_T + _EPSILON))


def kernel(context_embeddings, principals, action_flat, action_rowids):
    principals = jnp.squeeze(principals).reshape(_B, 1)
    rowids = action_rowids.reshape(1, _T)
    out = pl.pallas_call(
        _loss_kernel,
        in_specs=[
            pl.BlockSpec(memory_space=pltpu.MemorySpace.VMEM),
            pl.BlockSpec(memory_space=pltpu.MemorySpace.VMEM),
            pl.BlockSpec(memory_space=pltpu.MemorySpace.HBM),
            pl.BlockSpec(memory_space=pltpu.MemorySpace.VMEM),
        ],
        out_shape=jax.ShapeDtypeStruct((1, 1), jnp.float32),
        scratch_shapes=(
            [pltpu.VMEM((_T, _D), jnp.float32)]
            + [pltpu.SemaphoreType.DMA] * _NSTREAM
        ),
    )(context_embeddings, principals, action_flat, rowids)
    return out[0, 0]


# speedup vs baseline: 1.0579x; 1.0579x over previous
import jax
import jax.numpy as jnp
from jax import lax
from jax.experimental import pallas as pl
from jax.experimental.pallas import tpu as pltpu

_EPSILON = 1e-06
_SOFT_MARGIN = 0.5
_HARD_MARGIN = 0.1
_NUM_NEG = 8
_B = 16
_D = 128
_T = 4096
_NSTREAM = 4
_CH = _T // _NSTREAM


def _chunk_loss(ctx, prin, act, rid):
    cw = act.shape[0]
    logits = lax.dot_general(ctx, act,
                             dimension_numbers=(((1,), (1,)), ((), ())),
                             preferred_element_type=jnp.float32)
    logits = logits * (1.0 / (_D ** 0.5))
    sc = jax.nn.sigmoid(logits)
    s = -jnp.log(_EPSILON + 1.0 - sc)

    row = lax.broadcasted_iota(jnp.int32, (_B, cw), 0)
    off = (row - rid) & (_B - 1)

    pos = off == 0
    neg = (off >= 1) & (off <= _NUM_NEG)

    s_pos = jnp.sum(jnp.where(pos, s, 0.0), axis=0, keepdims=True)
    prin_q = jnp.sum(jnp.where(pos, prin, 0), axis=0, keepdims=True)
    w = jnp.where(neg & (prin != prin_q), 1.0, 0.0)

    x = s - s_pos + _HARD_MARGIN
    quad = x * x / (2.0 * _SOFT_MARGIN)
    lin = x - _SOFT_MARGIN / 2.0
    h = jnp.where(x <= 0.0, 0.0, jnp.where(x < _SOFT_MARGIN, quad, lin))

    num = jnp.sum(h * w, axis=0, keepdims=True)
    den = jnp.sum(w, axis=0, keepdims=True) + _EPSILON
    per_query = num / den
    return jnp.sum(per_query, axis=1, keepdims=True)


def _loss_kernel(ctx_ref, prin_ref, act_hbm, rowid_ref, out_ref,
                 act_vmem, *sems):
    copies = []
    for c in range(_NSTREAM):
        cp = pltpu.make_async_copy(
            act_hbm.at[pl.ds(c * _CH, _CH), :],
            act_vmem.at[pl.ds(c * _CH, _CH), :],
            sems[c])
        cp.start()
        copies.append(cp)

    ctx = ctx_ref[...]
    prin = prin_ref[...]
    for c in range(_NSTREAM):
        copies[c].wait()
    acc = _chunk_loss(ctx, prin, act_vmem[...], rowid_ref[...])
    out_ref[...] = acc * (1.0 / (_T + _EPSILON))


def kernel(context_embeddings, principals, action_flat, action_rowids):
    principals = jnp.squeeze(principals).reshape(_B, 1)
    rowids = action_rowids.reshape(1, _T)
    out = pl.pallas_call(
        _loss_kernel,
        in_specs=[
            pl.BlockSpec(memory_space=pltpu.MemorySpace.VMEM),
            pl.BlockSpec(memory_space=pltpu.MemorySpace.VMEM),
            pl.BlockSpec(memory_space=pltpu.MemorySpace.HBM),
            pl.BlockSpec(memory_space=pltpu.MemorySpace.VMEM),
        ],
        out_shape=jax.ShapeDtypeStruct((1, 1), jnp.float32),
        scratch_shapes=(
            [pltpu.VMEM((_T, _D), jnp.float32)]
            + [pltpu.SemaphoreType.DMA] * _NSTREAM
        ),
    )(context_embeddings, principals, action_flat, rowids)
    return out[0, 0]


# speedup vs baseline: 1.2711x; 1.2015x over previous
import jax
import jax.numpy as jnp
from jax import lax
from jax.experimental import pallas as pl

_EPSILON = 1e-06
_SOFT_MARGIN = 0.5
_HARD_MARGIN = 0.1
_NUM_NEG = 8
_B = 16
_D = 128
_T = 4096


_G = 1
_TB = _T // _G


def _loss_kernel(ctx_ref, prin_ref, actT_ref, rowid_ref, out_ref):
    logits = lax.dot_general(ctx_ref[...], actT_ref[...],
                             dimension_numbers=(((1,), (1,)), ((), ())),
                             preferred_element_type=jnp.float32)
    logits = logits * (1.0 / (_D ** 0.5))
    sc = jax.nn.sigmoid(logits)
    s = -jnp.log(_EPSILON + 1.0 - sc)

    row = lax.broadcasted_iota(jnp.int32, (_B, _TB), 0)
    rid = rowid_ref[...]
    off = (row - rid) & (_B - 1)

    pos = off == 0
    neg = (off >= 1) & (off <= _NUM_NEG)

    s_pos = jnp.sum(jnp.where(pos, s, 0.0), axis=0, keepdims=True)

    prin_row = prin_ref[...]
    prin_q = jnp.sum(jnp.where(pos, prin_row, 0), axis=0, keepdims=True)
    w = jnp.where(neg & (prin_row != prin_q), 1.0, 0.0)

    x = s - s_pos + _HARD_MARGIN
    quad = x * x / (2.0 * _SOFT_MARGIN)
    lin = x - _SOFT_MARGIN / 2.0
    h = jnp.where(x <= 0.0, 0.0, jnp.where(x < _SOFT_MARGIN, quad, lin))

    num = jnp.sum(h * w, axis=0, keepdims=True)
    den = jnp.sum(w, axis=0, keepdims=True) + _EPSILON
    per_query = num / den
    partial = jnp.sum(per_query, axis=1, keepdims=True) * (1.0 / (_T + _EPSILON))

    @pl.when(pl.program_id(0) == 0)
    def _init():
        out_ref[...] = partial

    @pl.when(pl.program_id(0) > 0)
    def _acc():
        out_ref[...] += partial


def kernel(context_embeddings, principals, action_flat, action_rowids):
    principals = jnp.squeeze(principals).reshape(_B, 1)
    actT = action_flat
    rowids = action_rowids.reshape(1, _T)
    out = pl.pallas_call(
        _loss_kernel,
        grid=(_G,),
        in_specs=[
            pl.BlockSpec((_B, _D), lambda i: (0, 0)),
            pl.BlockSpec((_B, 1), lambda i: (0, 0)),
            pl.BlockSpec((_TB, _D), lambda i: (i, 0)),
            pl.BlockSpec((1, _TB), lambda i: (0, i)),
        ],
        out_specs=pl.BlockSpec((1, 1), lambda i: (0, 0)),
        out_shape=jax.ShapeDtypeStruct((1, 1), jnp.float32),
    )(context_embeddings, principals, actT, rowids)
    return out[0, 0]


# speedup vs baseline: 1.8746x; 1.4749x over previous
import jax
import jax.numpy as jnp
from jax import lax
from jax.experimental import pallas as pl

_EPSILON = 1e-06
_SOFT_MARGIN = 0.5
_HARD_MARGIN = 0.1
_NUM_NEG = 8
_B = 16
_D = 128
_T = 4096


_G = 1
_TB = _T // _G


def _loss_kernel(ctx_ref, prin_ref, actT_ref, rowid_ref, out_ref):
    logits = lax.dot_general(ctx_ref[...], actT_ref[...],
                             dimension_numbers=(((1,), (1,)), ((), ())),
                             preferred_element_type=jnp.float32)
    logits = logits * (1.0 / (_D ** 0.5))
    sc = jax.nn.sigmoid(logits)
    s = -jnp.log(_EPSILON + 1.0 - sc)

    row = lax.broadcasted_iota(jnp.int32, (_B, _TB), 0)
    rid = rowid_ref[...]
    off = (row - rid) & (_B - 1)

    pos = off == 0
    neg = (off >= 1) & (off <= _NUM_NEG)

    s_pos = jnp.sum(jnp.where(pos, s, 0.0), axis=0, keepdims=True)

    prin_row = jnp.transpose(prin_ref[...])
    prin_q = jnp.sum(jnp.where(pos, prin_row, 0), axis=0, keepdims=True)
    w = jnp.where(neg & (prin_row != prin_q), 1.0, 0.0)

    x = s - s_pos + _HARD_MARGIN
    quad = x * x / (2.0 * _SOFT_MARGIN)
    lin = x - _SOFT_MARGIN / 2.0
    h = jnp.where(x <= 0.0, 0.0, jnp.where(x < _SOFT_MARGIN, quad, lin))

    num = jnp.sum(h * w, axis=0, keepdims=True)
    den = jnp.sum(w, axis=0, keepdims=True) + _EPSILON
    per_query = num / den
    partial = jnp.sum(per_query, axis=1, keepdims=True) * (1.0 / (_T + _EPSILON))

    @pl.when(pl.program_id(0) == 0)
    def _init():
        out_ref[...] = partial

    @pl.when(pl.program_id(0) > 0)
    def _acc():
        out_ref[...] += partial


def kernel(context_embeddings, principals, action_flat, action_rowids):
    principals = jnp.squeeze(principals).reshape(1, _B)
    actT = action_flat
    rowids = action_rowids.reshape(1, _T)
    out = pl.pallas_call(
        _loss_kernel,
        grid=(_G,),
        in_specs=[
            pl.BlockSpec((_B, _D), lambda i: (0, 0)),
            pl.BlockSpec((1, _B), lambda i: (0, 0)),
            pl.BlockSpec((_TB, _D), lambda i: (i, 0)),
            pl.BlockSpec((1, _TB), lambda i: (0, i)),
        ],
        out_specs=pl.BlockSpec((1, 1), lambda i: (0, 0)),
        out_shape=jax.ShapeDtypeStruct((1, 1), jnp.float32),
    )(context_embeddings, principals, actT, rowids)
    return out[0, 0]


# speedup vs baseline: 1.8915x; 1.0090x over previous
import jax
import jax.numpy as jnp
from jax import lax
from jax.experimental import pallas as pl

_EPSILON = 1e-06
_SOFT_MARGIN = 0.5
_HARD_MARGIN = 0.1
_NUM_NEG = 8
_B = 16
_D = 128
_T = 4096


_G = 1
_TB = _T // _G


def _loss_kernel(ctx_ref, prin_ref, actT_ref, rowid_ref, out_ref):
    logits = lax.dot_general(ctx_ref[...], actT_ref[...],
                             dimension_numbers=(((1,), (1,)), ((), ())),
                             preferred_element_type=jnp.float32)
    logits = logits * (1.0 / (_D ** 0.5))
    sc = jax.nn.sigmoid(logits)
    s = -jnp.log(_EPSILON + 1.0 - sc)

    row = lax.broadcasted_iota(jnp.int32, (_B, _TB), 0)
    rid = rowid_ref[...].reshape(1, _TB)
    off = (row - rid) & (_B - 1)

    pos = off == 0
    neg = (off >= 1) & (off <= _NUM_NEG)

    s_pos = jnp.sum(jnp.where(pos, s, 0.0), axis=0, keepdims=True)

    prin_row = jnp.transpose(prin_ref[...])
    prin_q = jnp.sum(jnp.where(pos, prin_row, 0), axis=0, keepdims=True)
    w = jnp.where(neg & (prin_row != prin_q), 1.0, 0.0)

    x = s - s_pos + _HARD_MARGIN
    quad = x * x / (2.0 * _SOFT_MARGIN)
    lin = x - _SOFT_MARGIN / 2.0
    h = jnp.where(x <= 0.0, 0.0, jnp.where(x < _SOFT_MARGIN, quad, lin))

    num = jnp.sum(h * w, axis=0, keepdims=True)
    den = jnp.sum(w, axis=0, keepdims=True) + _EPSILON
    per_query = num / den
    partial = jnp.sum(per_query, axis=1, keepdims=True) * (1.0 / (_T + _EPSILON))

    @pl.when(pl.program_id(0) == 0)
    def _init():
        out_ref[...] = partial

    @pl.when(pl.program_id(0) > 0)
    def _acc():
        out_ref[...] += partial


def kernel(context_embeddings, principals, action_flat, action_rowids):
    principals = jnp.squeeze(principals).reshape(1, _B)
    actT = action_flat
    rowids = action_rowids
    out = pl.pallas_call(
        _loss_kernel,
        grid=(_G,),
        in_specs=[
            pl.BlockSpec((_B, _D), lambda i: (0, 0)),
            pl.BlockSpec((1, _B), lambda i: (0, 0)),
            pl.BlockSpec((_TB, _D), lambda i: (i, 0)),
            pl.BlockSpec((_TB,), lambda i: (i,)),
        ],
        out_specs=pl.BlockSpec((1, 1), lambda i: (0, 0)),
        out_shape=jax.ShapeDtypeStruct((1, 1), jnp.float32),
    )(context_embeddings, principals, actT, rowids)
    return out[0, 0]
